# baseline (speedup 1.0000x reference)
import functools

import jax
import jax.numpy as jnp
from jax import lax
from jax.experimental import pallas as pl
from jax.experimental.pallas import tpu as pltpu
from jax.experimental.pallas import tpu_sc as plsc

NS = 16
NC = 2
LANES = 16
DH = 128
CHUNK = 80
NBUF = 4
BLK = 5


def _spmm_body(rowr, colr, valr, br, outr,
               idxc, idxr, valv, gbuf, acc,
               gsems, ssems, tsem, isem, zsem, wsem,
               *, n_rows, nchunk):
    c = lax.axis_index("c")
    s = lax.axis_index("s")
    nblk = nchunk // BLK
    nedge = nchunk * CHUNK
    blke = BLK * CHUNK
    zrows = n_rows // NS
    cvec = lax.broadcast(c, (LANES,))
    nmax = lax.broadcast(n_rows - 1, (LANES,))

    def start_ctable(blk):
        ring = lax.rem(blk, 2)
        pltpu.async_copy(colr.at[pl.ds(s * nedge + blk * blke, blke)],
                         idxc.at[ring], isem)

    def finish_ctable(ring):
        pltpu.make_async_copy(colr.at[pl.ds(0, blke)], idxc.at[0], isem).wait()

        def cxform(i, carry):
            sl = pl.ds(i * LANES, LANES)
            idxc[ring, sl] = jnp.minimum(idxc[ring, sl], nmax) * 2 + cvec
            return carry
        lax.fori_loop(0, blke // LANES, cxform, 0)

    def start_rvtable(blk):
        ring = lax.rem(blk, 2)
        base = s * nedge + blk * blke
        pltpu.async_copy(rowr.at[pl.ds(base, blke)], idxr.at[ring], tsem)
        pltpu.async_copy(valr.at[pl.ds(base, blke)], valv.at[ring], tsem)

    def finish_rvtable(ring):
        pltpu.make_async_copy(rowr.at[pl.ds(0, blke)], idxr.at[0], tsem).wait()
        pltpu.make_async_copy(valr.at[pl.ds(0, blke)], valv.at[0], tsem).wait()

        def rclamp(i, carry):
            sl = pl.ds(i * LANES, LANES)
            idxr[ring, sl] = jnp.minimum(idxr[ring, sl], nmax)
            return carry
        lax.fori_loop(0, blke // LANES, rclamp, 0)

    start_ctable(0)
    start_rvtable(0)
    zv = jnp.zeros((LANES,), jnp.float32)

    def zrow(i, carry):
        for j in range(DH // LANES):
            gbuf[0, i, pl.ds(j * LANES, LANES)] = zv
        return carry
    lax.fori_loop(0, CHUNK, zrow, 0)
    zbase = s * zrows
    zcps = []
    nfull = zrows // CHUNK
    for r in range(nfull):
        zcps.append(pltpu.async_copy(
            gbuf.at[0], acc.at[pl.ds(zbase + r * CHUNK, CHUNK), :], wsem))
    rem = zrows - nfull * CHUNK
    if rem:
        zcps.append(pltpu.async_copy(
            gbuf.at[0, pl.ds(0, rem), :],
            acc.at[pl.ds(zbase + nfull * CHUNK, rem), :], wsem))
    finish_ctable(0)
    for cp in zcps:
        cp.wait()
    plsc.subcore_barrier()

    def start_gather(bi, k):
        ring = lax.rem(lax.div(k, BLK), 2)
        kk = lax.rem(k, BLK)
        pltpu.async_copy(
            br.at[idxc.at[ring, pl.ds(kk * CHUNK, CHUNK)]], gbuf.at[bi],
            gsems.at[bi])

    def wait_gather(bi):
        pltpu.make_async_copy(br.at[idxc.at[0, pl.ds(0, CHUNK)]], gbuf.at[bi],
                              gsems.at[bi]).wait()

    def start_scatter(bi, ring, kk):
        pltpu.async_copy(gbuf.at[bi],
                         acc.at[idxr.at[ring, pl.ds(kk * CHUNK, CHUNK)]],
                         ssems.at[bi], add=True)

    def wait_scatter(bi):
        pltpu.make_async_copy(gbuf.at[bi], acc.at[idxr.at[0, pl.ds(0, CHUNK)]],
                              ssems.at[bi]).wait()

    def scale_chunk(bi, ring, kk):
        def group_body(g, carry):
            vvec = valv[ring, pl.ds(kk * CHUNK + g * LANES, LANES)]
            for l in range(LANES):
                vv = lax.broadcast(vvec[l], (LANES,))
                i = g * LANES + l
                for j in range(DH // LANES):
                    sl = pl.ds(j * LANES, LANES)
                    gbuf[bi, i, sl] = gbuf[bi, i, sl] * vv
            return carry
        lax.fori_loop(0, CHUNK // LANES, group_body, 0)

    def slot(k, carry):
        bi = lax.rem(k, NBUF)
        blk = lax.div(k, BLK)
        kk = lax.rem(k, BLK)
        ring = lax.rem(blk, 2)

        @pl.when(kk == 0)
        def _():
            finish_rvtable(ring)

        @pl.when((kk == 2) & (k + 3 < nchunk))
        def _():
            finish_ctable(lax.rem(blk + 1, 2))

        wait_gather(bi)
        scale_chunk(bi, ring, kk)
        start_scatter(bi, ring, kk)

        bnext = lax.rem(k + 3, NBUF)

        @pl.when(k >= 1)
        def _():
            wait_scatter(bnext)

        @pl.when(k + 3 < nchunk)
        def _():
            start_gather(bnext, k + 3)

        @pl.when((kk == 0) & (k < (nblk - 1) * BLK))
        def _():
            start_rvtable(blk + 1)

        @pl.when((kk == 4) & (k < (nblk - 2) * BLK + 4))
        def _():
            start_ctable(blk + 2)
        return carry

    start_ctable(1)
    for j in range(NBUF - 1):
        start_gather(j, j)
    lax.fori_loop(0, nchunk, slot, 0)
    wait_scatter((nchunk - 1) % NBUF)

    plsc.subcore_barrier()
    pltpu.sync_copy(acc.at[pl.ds(zbase, zrows), :],
                    outr.at[pl.ds(zbase, zrows), pl.ds(c * DH, DH)])


@jax.jit
def _spmm(row1, col1, values, bview):
    n_rows = bview.shape[0] // NC
    nchunk = values.shape[0] // (NS * CHUNK)
    mesh = plsc.VectorSubcoreMesh(core_axis_name="c", subcore_axis_name="s")
    body = functools.partial(_spmm_body, n_rows=n_rows, nchunk=nchunk)
    out = pl.kernel(
        body,
        out_type=jax.ShapeDtypeStruct((n_rows, NC * DH), jnp.float32),
        mesh=mesh,
        scratch_types=[
            pltpu.VMEM((2, BLK * CHUNK), jnp.int32),
            pltpu.VMEM((2, BLK * CHUNK), jnp.int32),
            pltpu.VMEM((2, BLK * CHUNK), jnp.float32),
            pltpu.VMEM((NBUF, CHUNK, DH), jnp.float32),
            pltpu.VMEM_SHARED((n_rows, DH), jnp.float32),
            pltpu.SemaphoreType.DMA((NBUF,)),
            pltpu.SemaphoreType.DMA((NBUF,)),
            pltpu.SemaphoreType.DMA,
            pltpu.SemaphoreType.DMA,
            pltpu.SemaphoreType.DMA,
            pltpu.SemaphoreType.DMA,
        ],
        compiler_params=pltpu.CompilerParams(use_tc_tiling_on_sc=False),
    )(row1, col1, values, bview)
    return out


def kernel(indices, values, shape, b):
    n_rows = b.shape[0]
    bview = b.reshape(n_rows * NC, DH)
    return _spmm(indices[0], indices[1], values, bview)


# speedup vs baseline: 1.0014x; 1.0014x over previous
import functools

import jax
import jax.numpy as jnp
from jax import lax
from jax.experimental import pallas as pl
from jax.experimental.pallas import tpu as pltpu
from jax.experimental.pallas import tpu_sc as plsc

NS = 16
NC = 2
LANES = 16
DH = 128
CHUNK = 80
NBUF = 4
BLK = 5


def _spmm_body(rowr, colr, valr, br, outr,
               idxc, idxr, valv, gbuf, acc,
               gsems, ssems, tsem, isem, zsem, wsem,
               *, n_rows, nchunk):
    c = lax.axis_index("c")
    s = lax.axis_index("s")
    nblk = nchunk // BLK
    nedge = nchunk * CHUNK
    blke = BLK * CHUNK
    zrows = n_rows // NS
    cvec = lax.broadcast(c, (LANES,))
    nmax = lax.broadcast(n_rows - 1, (LANES,))

    def start_ctable(blk):
        ring = lax.rem(blk, 2)
        pltpu.async_copy(colr.at[pl.ds(s * nedge + blk * blke, blke)],
                         idxc.at[ring], isem)

    def finish_ctable(ring):
        pltpu.make_async_copy(colr.at[pl.ds(0, blke)], idxc.at[0], isem).wait()

        def cxform(i, carry):
            sl = pl.ds(i * LANES, LANES)
            idxc[ring, sl] = jnp.minimum(idxc[ring, sl], nmax) * 2 + cvec
            return carry
        lax.fori_loop(0, blke // LANES, cxform, 0)

    def start_rvtable(blk):
        ring = lax.rem(blk, 2)
        base = s * nedge + blk * blke
        pltpu.async_copy(rowr.at[pl.ds(base, blke)], idxr.at[ring], tsem)
        pltpu.async_copy(valr.at[pl.ds(base, blke)], valv.at[ring], tsem)

    def finish_rvtable(ring):
        pltpu.make_async_copy(rowr.at[pl.ds(0, blke)], idxr.at[0], tsem).wait()
        pltpu.make_async_copy(valr.at[pl.ds(0, blke)], valv.at[0], tsem).wait()

        def rclamp(i, carry):
            sl = pl.ds(i * LANES, LANES)
            idxr[ring, sl] = jnp.minimum(idxr[ring, sl], nmax)
            return carry
        lax.fori_loop(0, blke // LANES, rclamp, 0)

    start_ctable(0)
    start_rvtable(0)
    zv = jnp.zeros((LANES,), jnp.float32)

    def zrow(i, carry):
        for j in range(DH // LANES):
            gbuf[0, i, pl.ds(j * LANES, LANES)] = zv
        return carry
    lax.fori_loop(0, CHUNK, zrow, 0)
    zbase = s * zrows
    zcps = []
    nfull = zrows // CHUNK
    for r in range(nfull):
        zcps.append(pltpu.async_copy(
            gbuf.at[0], acc.at[pl.ds(zbase + r * CHUNK, CHUNK), :], wsem))
    rem = zrows - nfull * CHUNK
    if rem:
        zcps.append(pltpu.async_copy(
            gbuf.at[0, pl.ds(0, rem), :],
            acc.at[pl.ds(zbase + nfull * CHUNK, rem), :], wsem))
    finish_ctable(0)
    start_ctable(1)
    for cp in zcps:
        cp.wait()
    plsc.subcore_barrier()

    def start_gather(bi, k):
        ring = lax.rem(lax.div(k, BLK), 2)
        kk = lax.rem(k, BLK)
        pltpu.async_copy(
            br.at[idxc.at[ring, pl.ds(kk * CHUNK, CHUNK)]], gbuf.at[bi],
            gsems.at[bi])

    def wait_gather(bi):
        pltpu.make_async_copy(br.at[idxc.at[0, pl.ds(0, CHUNK)]], gbuf.at[bi],
                              gsems.at[bi]).wait()

    def start_scatter(bi, ring, kk):
        pltpu.async_copy(gbuf.at[bi],
                         acc.at[idxr.at[ring, pl.ds(kk * CHUNK, CHUNK)]],
                         ssems.at[bi], add=True)

    def wait_scatter(bi):
        pltpu.make_async_copy(gbuf.at[bi], acc.at[idxr.at[0, pl.ds(0, CHUNK)]],
                              ssems.at[bi]).wait()

    def scale_chunk(bi, ring, kk):
        def group_body(g, carry):
            vvec = valv[ring, pl.ds(kk * CHUNK + g * LANES, LANES)]
            for l in range(LANES):
                vv = lax.broadcast(vvec[l], (LANES,))
                i = g * LANES + l
                for j in range(DH // LANES):
                    sl = pl.ds(j * LANES, LANES)
                    gbuf[bi, i, sl] = gbuf[bi, i, sl] * vv
            return carry
        lax.fori_loop(0, CHUNK // LANES, group_body, 0)

    def slot(k, carry):
        bi = lax.rem(k, NBUF)
        blk = lax.div(k, BLK)
        kk = lax.rem(k, BLK)
        ring = lax.rem(blk, 2)

        @pl.when(kk == 0)
        def _():
            finish_rvtable(ring)

        @pl.when((kk == 3) & (k + 2 < nchunk))
        def _():
            finish_ctable(lax.rem(blk + 1, 2))

        wait_gather(bi)
        scale_chunk(bi, ring, kk)
        start_scatter(bi, ring, kk)

        bnext = lax.rem(k + 2, NBUF)

        @pl.when(k >= 2)
        def _():
            wait_scatter(bnext)

        @pl.when(k + 2 < nchunk)
        def _():
            start_gather(bnext, k + 2)

        @pl.when((kk == 0) & (k < (nblk - 1) * BLK))
        def _():
            start_rvtable(blk + 1)

        @pl.when((kk == 4) & (k < (nblk - 2) * BLK + 4))
        def _():
            start_ctable(blk + 2)
        return carry

    for j in range(2):
        start_gather(j, j)
    lax.fori_loop(0, nchunk, slot, 0)
    wait_scatter((nchunk - 2) % NBUF)
    wait_scatter((nchunk - 1) % NBUF)

    plsc.subcore_barrier()
    pltpu.sync_copy(acc.at[pl.ds(zbase, zrows), :],
                    outr.at[pl.ds(zbase, zrows), pl.ds(c * DH, DH)])


@jax.jit
def _spmm(row1, col1, values, bview):
    n_rows = bview.shape[0] // NC
    nchunk = values.shape[0] // (NS * CHUNK)
    mesh = plsc.VectorSubcoreMesh(core_axis_name="c", subcore_axis_name="s")
    body = functools.partial(_spmm_body, n_rows=n_rows, nchunk=nchunk)
    out = pl.kernel(
        body,
        out_type=jax.ShapeDtypeStruct((n_rows, NC * DH), jnp.float32),
        mesh=mesh,
        scratch_types=[
            pltpu.VMEM((2, BLK * CHUNK), jnp.int32),
            pltpu.VMEM((2, BLK * CHUNK), jnp.int32),
            pltpu.VMEM((2, BLK * CHUNK), jnp.float32),
            pltpu.VMEM((NBUF, CHUNK, DH), jnp.float32),
            pltpu.VMEM_SHARED((n_rows, DH), jnp.float32),
            pltpu.SemaphoreType.DMA((NBUF,)),
            pltpu.SemaphoreType.DMA((NBUF,)),
            pltpu.SemaphoreType.DMA,
            pltpu.SemaphoreType.DMA,
            pltpu.SemaphoreType.DMA,
            pltpu.SemaphoreType.DMA,
        ],
        compiler_params=pltpu.CompilerParams(use_tc_tiling_on_sc=False),
    )(row1, col1, values, bview)
    return out


def kernel(indices, values, shape, b):
    n_rows = b.shape[0]
    bview = b.reshape(n_rows * NC, DH)
    return _spmm(indices[0], indices[1], values, bview)


# speedup vs baseline: 2.0003x; 1.9974x over previous
import functools

import jax
import jax.numpy as jnp
from jax import lax
from jax.experimental import pallas as pl
from jax.experimental.pallas import tpu as pltpu
from jax.experimental.pallas import tpu_sc as plsc

NS = 16
NC = 2
LANES = 16
DH = 128
CHUNK = 80
NBUF = 3
BLK = 25


def _spmm_body(rowr, colr, valr, br, outr,
               idxc, idxr, valv, gbuf, acc,
               g0, g1, g2, s0, s1, s2, tsem, zsem,
               *, n_rows, nchunk):
    c = lax.axis_index("c")
    s = lax.axis_index("s")
    gsem = (g0, g1, g2)
    ssem = (s0, s1, s2)
    nblk = nchunk // BLK
    zrows = n_rows // NS

    nedge = nchunk * CHUNK
    blke = BLK * CHUNK
    cp_idxc = pltpu.async_copy(colr.at[pl.ds(s * nedge, nedge)], idxc, zsem)
    pltpu.async_copy(rowr.at[pl.ds(s * nedge, blke)], idxr.at[0], tsem)
    pltpu.async_copy(valr.at[pl.ds(s * nedge, blke)], valv.at[0], tsem)

    zv = jnp.zeros((LANES,), jnp.float32)

    def zrow(i, carry):
        for j in range(DH // LANES):
            gbuf[0, i, pl.ds(j * LANES, LANES)] = zv
        return carry
    lax.fori_loop(0, CHUNK, zrow, 0)
    zbase = s * zrows
    zcps = []
    nfull = zrows // CHUNK
    for r in range(nfull):
        zcps.append(pltpu.async_copy(
            gbuf.at[0], acc.at[pl.ds(zbase + r * CHUNK, CHUNK), :], s0))
    rem = zrows - nfull * CHUNK
    if rem:
        zcps.append(pltpu.async_copy(
            gbuf.at[0, pl.ds(0, rem), :],
            acc.at[pl.ds(zbase + nfull * CHUNK, rem), :], s0))

    cp_idxc.wait()
    cvec = lax.broadcast(c, (LANES,))

    def cxform(i, carry):
        sl = pl.ds(i * LANES, LANES)
        idxc[sl] = idxc[sl] * 2 + cvec
        return carry
    lax.fori_loop(0, nedge // LANES, cxform, 0)

    for cp in zcps:
        cp.wait()
    plsc.subcore_barrier()

    dcol = c * DH

    def start_gather(bi, k):
        pltpu.async_copy(br.at[idxc.at[pl.ds(k * CHUNK, CHUNK)]], gbuf.at[bi],
                         gsem[bi])

    def wait_gather(bi, k):
        pltpu.make_async_copy(br.at[idxc.at[pl.ds(k * CHUNK, CHUNK)]],
                              gbuf.at[bi], gsem[bi]).wait()

    def start_scatter(bi, ring, kk):
        pltpu.async_copy(gbuf.at[bi],
                         acc.at[idxr.at[ring, pl.ds(kk * CHUNK, CHUNK)]],
                         ssem[bi], add=True)

    def wait_scatter(bi):
        pltpu.make_async_copy(gbuf.at[bi], acc.at[idxr.at[0, pl.ds(0, CHUNK)]],
                              ssem[bi]).wait()

    def wait_table():
        pltpu.make_async_copy(rowr.at[pl.ds(0, blke)], idxr.at[0],
                              tsem).wait()
        pltpu.make_async_copy(valr.at[pl.ds(0, blke)], valv.at[0], tsem).wait()

    def start_table(blk):
        ring = lax.rem(blk, 2)
        base = s * nedge + blk * blke
        pltpu.async_copy(rowr.at[pl.ds(base, blke)], idxr.at[ring], tsem)
        pltpu.async_copy(valr.at[pl.ds(base, blke)], valv.at[ring], tsem)

    def scale_chunk(bi, ring, kk):
        def group_body(g, carry):
            vvec = valv[ring, pl.ds(kk * CHUNK + g * LANES, LANES)]
            for l in range(LANES):
                vv = lax.broadcast(vvec[l], (LANES,))
                i = g * LANES + l
                for j in range(DH // LANES):
                    sl = pl.ds(j * LANES, LANES)
                    gbuf[bi, i, sl] = gbuf[bi, i, sl] * vv
            return carry
        lax.fori_loop(0, CHUNK // LANES, group_body, 0, unroll=True)

    def slot(bi, k, t, guard_first, tail):
        blk = lax.div(k, BLK)
        kk = lax.rem(k, BLK)
        ring = lax.rem(blk, 2)

        @pl.when(kk == 0)
        def _():
            wait_table()

        wait_gather(bi, k)
        scale_chunk(bi, ring, kk)
        start_scatter(bi, ring, kk)

        bnext = (bi + 2) % NBUF
        if guard_first:
            @pl.when(t >= 1)
            def _():
                wait_scatter(bnext)
        else:
            wait_scatter(bnext)
        if not tail:
            start_gather(bnext, k + 2)

        @pl.when((kk == 0) & (k < (nblk - 1) * BLK))
        def _():
            start_table(blk + 1)

    start_gather(0, 0)
    start_gather(1, 1)

    nmain = (nchunk - 2) // NBUF

    def iter_body(t, carry):
        for bi in range(NBUF):
            slot(bi, NBUF * t + bi, t, bi == 0, False)
        return carry
    lax.fori_loop(0, nmain, iter_body, 0)

    slot((nchunk - 2) % NBUF, nchunk - 2, nmain, False, True)
    slot((nchunk - 1) % NBUF, nchunk - 1, nmain, False, True)
    wait_scatter((nchunk - 1) % NBUF)

    plsc.subcore_barrier()
    pltpu.sync_copy(acc.at[pl.ds(zbase, zrows), :],
                    outr.at[pl.ds(zbase, zrows), pl.ds(dcol, DH)])


@jax.jit
def _spmm(row1, col1, values, bview):
    n_rows = bview.shape[0] // NC
    nchunk = values.shape[0] // (NS * CHUNK)
    mesh = plsc.VectorSubcoreMesh(core_axis_name="c", subcore_axis_name="s")
    body = functools.partial(_spmm_body, n_rows=n_rows, nchunk=nchunk)
    out = pl.kernel(
        body,
        out_type=jax.ShapeDtypeStruct((n_rows, NC * DH), jnp.float32),
        mesh=mesh,
        scratch_types=[
            pltpu.VMEM((nchunk * CHUNK,), jnp.int32),
            pltpu.VMEM((2, BLK * CHUNK), jnp.int32),
            pltpu.VMEM((2, BLK * CHUNK), jnp.float32),
            pltpu.VMEM((NBUF, CHUNK, DH), jnp.float32),
            pltpu.VMEM_SHARED((n_rows, DH), jnp.float32),
            pltpu.SemaphoreType.DMA,
            pltpu.SemaphoreType.DMA,
            pltpu.SemaphoreType.DMA,
            pltpu.SemaphoreType.DMA,
            pltpu.SemaphoreType.DMA,
            pltpu.SemaphoreType.DMA,
            pltpu.SemaphoreType.DMA,
            pltpu.SemaphoreType.DMA,
        ],
        compiler_params=pltpu.CompilerParams(use_tc_tiling_on_sc=False),
    )(row1, col1, values, bview)
    return out


def kernel(indices, values, shape, b):
    n_rows = b.shape[0]
    bview = b.reshape(n_rows * NC, DH)
    return _spmm(indices[0], indices[1], values, bview)


# speedup vs baseline: 2.6683x; 1.3340x over previous
import functools

import jax
import jax.numpy as jnp
from jax import lax
from jax.experimental import pallas as pl
from jax.experimental.pallas import tpu as pltpu
from jax.experimental.pallas import tpu_sc as plsc

NS = 16
NC = 2
LANES = 16
DH = 128
CHUNK = 80
NBUF = 3
BLK = 25


def _spmm_body(rowr, colr, valr, br, outr,
               idxc, idxr, valv, gbuf, acc,
               g0, g1, g2, s0, s1, s2, tsem, zsem,
               *, n_rows, nchunk):
    c = lax.axis_index("c")
    s = lax.axis_index("s")
    gsem = (g0, g1, g2)
    ssem = (s0, s1, s2)
    nblk = nchunk // BLK
    zrows = n_rows // NS

    nedge = nchunk * CHUNK
    blke = BLK * CHUNK
    cp_idxc = pltpu.async_copy(colr.at[pl.ds(s * nedge, nedge)], idxc, zsem)
    pltpu.async_copy(rowr.at[pl.ds(s * nedge, blke)], idxr.at[0], tsem)
    pltpu.async_copy(valr.at[pl.ds(s * nedge, blke)], valv.at[0], tsem)

    zv = jnp.zeros((LANES,), jnp.float32)

    def zrow(i, carry):
        for j in range(DH // LANES):
            gbuf[0, i, pl.ds(j * LANES, LANES)] = zv
        return carry
    lax.fori_loop(0, CHUNK, zrow, 0)
    zbase = s * zrows
    zcps = []
    nfull = zrows // CHUNK
    for r in range(nfull):
        zcps.append(pltpu.async_copy(
            gbuf.at[0], acc.at[pl.ds(zbase + r * CHUNK, CHUNK), :], s0))
    rem = zrows - nfull * CHUNK
    if rem:
        zcps.append(pltpu.async_copy(
            gbuf.at[0, pl.ds(0, rem), :],
            acc.at[pl.ds(zbase + nfull * CHUNK, rem), :], s0))

    cp_idxc.wait()
    cvec = lax.broadcast(c, (LANES,))

    def cxform(i, carry):
        sl = pl.ds(i * LANES, LANES)
        idxc[sl] = idxc[sl] * 2 + cvec
        return carry
    lax.fori_loop(0, nedge // LANES, cxform, 0)

    for cp in zcps:
        cp.wait()
    plsc.subcore_barrier()

    dcol = c * DH

    def start_gather(bi, k):
        pltpu.async_copy(br.at[idxc.at[pl.ds(k * CHUNK, CHUNK)]], gbuf.at[bi],
                         gsem[bi])

    def wait_gather(bi, k):
        pltpu.make_async_copy(br.at[idxc.at[pl.ds(k * CHUNK, CHUNK)]],
                              gbuf.at[bi], gsem[bi]).wait()

    def start_scatter(bi, ring, kk):
        pltpu.async_copy(gbuf.at[bi],
                         acc.at[idxr.at[ring, pl.ds(kk * CHUNK, CHUNK)]],
                         ssem[bi], add=True)

    def wait_scatter(bi):
        pltpu.make_async_copy(gbuf.at[bi], acc.at[idxr.at[0, pl.ds(0, CHUNK)]],
                              ssem[bi]).wait()

    def wait_table():
        pltpu.make_async_copy(rowr.at[pl.ds(0, blke)], idxr.at[0],
                              tsem).wait()
        pltpu.make_async_copy(valr.at[pl.ds(0, blke)], valv.at[0], tsem).wait()

    def start_table(blk):
        ring = lax.rem(blk, 2)
        base = s * nedge + blk * blke
        pltpu.async_copy(rowr.at[pl.ds(base, blke)], idxr.at[ring], tsem)
        pltpu.async_copy(valr.at[pl.ds(base, blke)], valv.at[ring], tsem)

    def scale_chunk(bi, ring, kk):
        def group_body(g, carry):
            vvec = valv[ring, pl.ds(kk * CHUNK + g * LANES, LANES)]
            for l in range(LANES):
                vv = lax.broadcast(vvec[l], (LANES,))
                i = g * LANES + l
                for j in range(DH // LANES):
                    sl = pl.ds(j * LANES, LANES)
                    gbuf[bi, i, sl] = gbuf[bi, i, sl] * vv
            return carry
        lax.fori_loop(0, CHUNK // LANES, group_body, 0)

    def slot(bi, k, t, guard_first, tail):
        blk = lax.div(k, BLK)
        kk = lax.rem(k, BLK)
        ring = lax.rem(blk, 2)

        @pl.when(kk == 0)
        def _():
            wait_table()

        wait_gather(bi, k)
        scale_chunk(bi, ring, kk)
        start_scatter(bi, ring, kk)

        bnext = (bi + 2) % NBUF
        if guard_first:
            @pl.when(t >= 1)
            def _():
                wait_scatter(bnext)
        else:
            wait_scatter(bnext)
        if not tail:
            start_gather(bnext, k + 2)

        @pl.when((kk == 0) & (k < (nblk - 1) * BLK))
        def _():
            start_table(blk + 1)

    start_gather(0, 0)
    start_gather(1, 1)

    nmain = (nchunk - 2) // NBUF

    def iter_body(t, carry):
        for bi in range(NBUF):
            slot(bi, NBUF * t + bi, t, bi == 0, False)
        return carry
    lax.fori_loop(0, nmain, iter_body, 0)

    slot((nchunk - 2) % NBUF, nchunk - 2, nmain, False, True)
    slot((nchunk - 1) % NBUF, nchunk - 1, nmain, False, True)
    wait_scatter((nchunk - 1) % NBUF)

    plsc.subcore_barrier()
    pltpu.sync_copy(acc.at[pl.ds(zbase, zrows), :],
                    outr.at[pl.ds(zbase, zrows), pl.ds(dcol, DH)])


@jax.jit
def _spmm(row1, col1, values, bview):
    n_rows = bview.shape[0] // NC
    nchunk = values.shape[0] // (NS * CHUNK)
    mesh = plsc.VectorSubcoreMesh(core_axis_name="c", subcore_axis_name="s")
    body = functools.partial(_spmm_body, n_rows=n_rows, nchunk=nchunk)
    out = pl.kernel(
        body,
        out_type=jax.ShapeDtypeStruct((n_rows, NC * DH), jnp.float32),
        mesh=mesh,
        scratch_types=[
            pltpu.VMEM((nchunk * CHUNK,), jnp.int32),
            pltpu.VMEM((2, BLK * CHUNK), jnp.int32),
            pltpu.VMEM((2, BLK * CHUNK), jnp.float32),
            pltpu.VMEM((NBUF, CHUNK, DH), jnp.float32),
            pltpu.VMEM_SHARED((n_rows, DH), jnp.float32),
            pltpu.SemaphoreType.DMA,
            pltpu.SemaphoreType.DMA,
            pltpu.SemaphoreType.DMA,
            pltpu.SemaphoreType.DMA,
            pltpu.SemaphoreType.DMA,
            pltpu.SemaphoreType.DMA,
            pltpu.SemaphoreType.DMA,
            pltpu.SemaphoreType.DMA,
        ],
        compiler_params=pltpu.CompilerParams(use_tc_tiling_on_sc=False),
    )(row1, col1, values, bview)
    return out


def kernel(indices, values, shape, b):
    n_rows = b.shape[0]
    bview = b.reshape(n_rows * NC, DH)
    return _spmm(indices[0], indices[1], values, bview)
